# baseline (speedup 1.0000x reference)
import functools

import numpy as np
import jax
import jax.numpy as jnp
from jax import lax
from jax.experimental import pallas as pl
from jax.experimental.pallas import tpu as pltpu
from jax.experimental.pallas import tpu_sc as plsc

H = W = 224
N = H * W
W2 = W // 2 + 1
N2 = H * W2
KR = 8
NBLK = N // 512
NBP = 112
CB = 8
SUP = 7168
NSUP = N // SUP
F32 = jnp.float32
I32 = jnp.int32


def _dft_mats():
    k = np.arange(H, dtype=np.float64)
    ang = -2.0 * np.pi * np.outer(k, k) / H
    return np.cos(ang).astype(np.float32), np.sin(ang).astype(np.float32)


def _perm_unshifted():
    cy, cx = H // 2, W // 2
    y = np.arange(H, dtype=np.float32) - cy
    x = np.arange(W, dtype=np.float32) - cx
    yy, xx = np.meshgrid(y, x, indexing="ij")
    dist = np.sqrt(xx * xx + yy * yy).astype(np.float32)
    sorted_idx = np.argsort(dist.reshape(-1), kind="stable")
    r, c = sorted_idx // W, sorted_idx % W
    ru, cu = (r + cy) % H, (c + cx) % W
    mr = np.where(cu <= W // 2, ru, (H - ru) % H)
    mc = np.where(cu <= W // 2, cu, W - cu)
    return (mr * W2 + mc).astype(np.int32)


_CM, _SM = _dft_mats()
_PERM = _perm_unshifted()


def _tc_body(x_ref, cm_ref, sm_ref, chh_ref, shh_ref, summag_ref,
             chansum_ref):
    j = pl.program_id(1)

    @pl.when(j == 0)
    def _init():
        summag_ref[...] = jnp.zeros_like(summag_ref)
        chansum_ref[...] = jnp.zeros_like(chansum_ref)

    cm = cm_ref[...]
    sm = sm_ref[...]
    chh = chh_ref[...]
    shh = shh_ref[...]
    sm_acc = jnp.zeros((H, W2), F32)
    cs_acc = jnp.zeros((H, W2), F32)
    dot = functools.partial(jnp.dot, preferred_element_type=F32,
                            precision=lax.Precision.HIGHEST)
    for c in range(CB):
        x = x_ref[0, c]
        p = dot(x, chh)
        q = dot(x, shh)
        re = dot(cm, p) - dot(sm, q)
        im = dot(cm, q) + dot(sm, p)
        mag = jnp.sqrt(re * re + im * im)
        sm_acc = sm_acc + mag
        cs_acc = cs_acc + jnp.log(1.0 + mag)
    summag_ref[0] += sm_acc
    chansum_ref[0] += cs_acc


def _tc_stage(x4, cm, sm, chh, shh):
    grid = (4, 96 // CB)
    return pl.pallas_call(
        _tc_body,
        grid=grid,
        in_specs=[
            pl.BlockSpec((1, CB, H, W), lambda s, j: (s, j, 0, 0)),
            pl.BlockSpec((H, W), lambda s, j: (0, 0)),
            pl.BlockSpec((H, W), lambda s, j: (0, 0)),
            pl.BlockSpec((H, W2), lambda s, j: (0, 0)),
            pl.BlockSpec((H, W2), lambda s, j: (0, 0)),
        ],
        out_specs=[
            pl.BlockSpec((1, H, W2), lambda s, j: (s, 0, 0)),
            pl.BlockSpec((1, H, W2), lambda s, j: (s, 0, 0)),
        ],
        out_shape=[
            jax.ShapeDtypeStruct((4, H, W2), F32),
            jax.ShapeDtypeStruct((4, H, W2), F32),
        ],
        compiler_params=pltpu.CompilerParams(
            dimension_semantics=("arbitrary", "arbitrary")),
    )(x4, cm, sm, chh, shh)


def _lane_iota():
    return lax.iota(I32, 16)


def _splat(x):
    return jnp.full((16,), x)


def _lane_sum(v):
    return plsc.cumsum(v)[15]


def _sc_body(data_hbm, perm_hbm, out_hbm,
             sorted_sh, part_sh,
             src_v, idx_v, sbuf_v, lp_v,
             p_cf, p_ca, p_mf, p_ma,
             bcf_v, bca_v, bavg_v, cum_v,
             blkA_v, blkB_v, blkC_v, blkD_v, outv_v):
    core = lax.axis_index("c")
    sub = lax.axis_index("s")
    iota = _lane_iota()
    zero16 = jnp.zeros((16,), F32)
    scale_v = jnp.full((16,), F32(0.5 / 96.0))

    @pl.when(sub < 8)
    def _phase_a():
        samp_loc = sub // 4
        a = sub % 4
        s_glob = 2 * (core * 2 + samp_loc) + (a % 2)
        row = jnp.where(a < 2, s_glob, 8 + s_glob)
        lr = samp_loc * 4 + a
        pltpu.sync_copy(data_hbm.at[row], src_v)

        def super_body(sc_i, carry):
            pltpu.sync_copy(perm_hbm.at[pl.ds(sc_i * SUP, SUP)], idx_v)

            def blk_body(jb, carry2):
                def vec_body(i, acc):
                    off = jb * 512 + i * 16
                    iv = idx_v[pl.ds(off, 16)]
                    v = plsc.load_gather(src_v, [iv])
                    sbuf_v[pl.ds(off, 16)] = v
                    return acc + v
                acc = lax.fori_loop(0, 32, vec_body, zero16)
                lp_v[pl.ds((sc_i * 14 + jb) * 16, 16)] = acc
                return carry2
            lax.fori_loop(0, 14, blk_body, 0)
            pltpu.sync_copy(sbuf_v, sorted_sh.at[lr, pl.ds(sc_i * SUP, SUP)])
            return carry
        lax.fori_loop(0, NSUP, super_body, 0)
        for q in range(16 * NBLK, 16 * NBP, 16):
            lp_v[pl.ds(q, 16)] = zero16
        pltpu.sync_copy(lp_v, part_sh.at[lr])

    plsc.subcore_barrier()

    @pl.when(sub < 2)
    def _phase_b():
        samp_loc = sub
        base = samp_loc * 4
        pltpu.sync_copy(part_sh.at[base + 0], p_cf)
        pltpu.sync_copy(part_sh.at[base + 1], p_ca)
        pltpu.sync_copy(part_sh.at[base + 2], p_mf)
        pltpu.sync_copy(part_sh.at[base + 3], p_ma)

        def bs_vec(p_ref, g):
            bidx = (g * 16 + iota) * 16
            acc = zero16
            for l in range(16):
                acc = acc + plsc.load_gather(p_ref, [bidx + l])
            return acc

        for g in range(7):
            bcf_v[pl.ds(g * 16, 16)] = bs_vec(p_cf, g)
            bca_v[pl.ds(g * 16, 16)] = bs_vec(p_ca, g)
            bavg_v[pl.ds(g * 16, 16)] = (
                (bs_vec(p_mf, g) + bs_vec(p_ma, g)) * scale_v)

        carry_v = zero16
        for g in range(7):
            pf = carry_v + plsc.cumsum(bavg_v[pl.ds(g * 16, 16)])
            cum_v[pl.ds(g * 16, 16)] = pf
            carry_v = _splat(pf[15])
        tot_s = cum_v[pl.ds(96, 16)][NBLK - 1 - 96]
        tgt_v = (jnp.full((16,), tot_s) + F32(1e-12)) / F32(KR)

        ends = []
        start_s = jnp.int32(0)
        for kk in range(KR):
            tc_v = F32(kk + 1) * tgt_v
            cnt_s = jnp.int32(0)
            for g in range(7):
                lt = cum_v[pl.ds(g * 16, 16)] < tc_v
                cnt_s = cnt_s + plsc.all_reduce_population_count(lt)[0]
            jb = cnt_s
            jbc = jnp.minimum(jb, NBLK - 1)
            pltpu.sync_copy(sorted_sh.at[base + 2, pl.ds(jbc * 512, 512)],
                            blkA_v)
            pltpu.sync_copy(sorted_sh.at[base + 3, pl.ds(jbc * 512, 512)],
                            blkB_v)
            jb_v = _splat(jb)
            pref_acc = zero16
            for g in range(7):
                jv = _splat(g * 16) + iota
                pref_acc = pref_acc + jnp.where(
                    jv < jb_v, bavg_v[pl.ds(g * 16, 16)], 0.0)
            pref_v = _splat(_lane_sum(pref_acc))

            def rbody(i, c2):
                cv, pv = c2
                w = (blkA_v[pl.ds(i * 16, 16)]
                     + blkB_v[pl.ds(i * 16, 16)]) * scale_v
                pf = cv + plsc.cumsum(w)
                pv = pv + jnp.where(pf < tc_v, 1, 0)
                return (_splat(pf[15]), pv)
            _, pos_v = lax.fori_loop(0, 32, rbody,
                                     (pref_v, jnp.zeros((16,), I32)))
            pos_s = _lane_sum_i32(pos_v)
            end_s = jnp.where(jb >= NBLK, jnp.int32(N),
                              jb * 512 + pos_s)
            end_s = jnp.minimum(jnp.maximum(end_s, start_s + 1),
                                jnp.int32(N))
            if kk == KR - 1:
                end_s = jnp.int32(N)
            ends.append(end_s)
            start_s = end_s

        num_v = zero16
        den_v = zero16
        prev_s = jnp.int32(0)
        for kk in range(KR):
            e_s = ends[kk]
            cnt_i = e_s - prev_s
            jb0 = prev_s // 512
            r0 = prev_s - jb0 * 512
            jb1 = e_s // 512
            r1 = e_s - jb1 * 512
            jb0c = jnp.minimum(jb0, NBLK - 1)
            jb1c = jnp.minimum(jb1, NBLK - 1)
            pltpu.sync_copy(sorted_sh.at[base + 0, pl.ds(jb0c * 512, 512)],
                            blkA_v)
            pltpu.sync_copy(sorted_sh.at[base + 0, pl.ds(jb1c * 512, 512)],
                            blkB_v)
            pltpu.sync_copy(sorted_sh.at[base + 1, pl.ds(jb0c * 512, 512)],
                            blkC_v)
            pltpu.sync_copy(sorted_sh.at[base + 1, pl.ds(jb1c * 512, 512)],
                            blkD_v)
            nsame_v = _splat(jb1) > _splat(jb0)
            r0_v = _splat(r0)
            r1_v = _splat(r1)

            def ebody2(i, accs):
                aLf, aRf, aLa, aRa = accs
                li = _splat(i) * 16 + iota
                lm = (li >= r0_v) & (nsame_v | (li < r1_v))
                rm = (li < r1_v) & nsame_v
                aLf = aLf + jnp.where(lm, blkA_v[pl.ds(i * 16, 16)], 0.0)
                aRf = aRf + jnp.where(rm, blkB_v[pl.ds(i * 16, 16)], 0.0)
                aLa = aLa + jnp.where(lm, blkC_v[pl.ds(i * 16, 16)], 0.0)
                aRa = aRa + jnp.where(rm, blkD_v[pl.ds(i * 16, 16)], 0.0)
                return (aLf, aRf, aLa, aRa)
            aLf, aRf, aLa, aRa = lax.fori_loop(
                0, 32, ebody2, (zero16, zero16, zero16, zero16))

            jb0_v = _splat(jb0)
            jb1_v = _splat(jb1)
            midf = zero16
            mida = zero16
            for g in range(7):
                jv = _splat(g * 16) + iota
                mm = (jv > jb0_v) & (jv < jb1_v)
                midf = midf + jnp.where(mm, bcf_v[pl.ds(g * 16, 16)], 0.0)
                mida = mida + jnp.where(mm, bca_v[pl.ds(g * 16, 16)], 0.0)

            rsf_s = _lane_sum(aLf + aRf + midf)
            rsa_s = _lane_sum(aLa + aRa + mida)
            cnt_v = _splat(cnt_i)
            denom_v = F32(96.0) * jnp.maximum(cnt_v, 1).astype(F32)
            p_v = jnp.full((16,), rsf_s) / denom_v
            pa_v = jnp.full((16,), rsa_s) / denom_v
            valid = cnt_v > 0
            d_v = p_v - pa_v
            num_v = num_v + jnp.where(valid, d_v * d_v, 0.0)
            den_v = den_v + jnp.where(valid, 1.0, 0.0)
            prev_s = e_s

        outv_v[...] = num_v / den_v
        pltpu.sync_copy(
            outv_v, out_hbm.at[pl.ds((core * 2 + samp_loc) * 16, 16)])


def _lane_sum_i32(v):
    return plsc.cumsum(v)[15]


def _sc_stage(data, perm):
    mesh = plsc.VectorSubcoreMesh(core_axis_name="c", subcore_axis_name="s")
    kfn = pl.kernel(
        _sc_body,
        mesh=mesh,
        out_type=jax.ShapeDtypeStruct((64,), F32),
        compiler_params=pltpu.CompilerParams(needs_layout_passes=False),
        scratch_types=[
            pltpu.VMEM_SHARED((8, N), F32),
            pltpu.VMEM_SHARED((8, 16 * NBP), F32),
            pltpu.VMEM((N2,), F32),
            pltpu.VMEM((SUP,), I32),
            pltpu.VMEM((SUP,), F32),
            pltpu.VMEM((16 * NBP,), F32),
            pltpu.VMEM((16 * NBP,), F32),
            pltpu.VMEM((16 * NBP,), F32),
            pltpu.VMEM((16 * NBP,), F32),
            pltpu.VMEM((16 * NBP,), F32),
            pltpu.VMEM((NBP,), F32),
            pltpu.VMEM((NBP,), F32),
            pltpu.VMEM((NBP,), F32),
            pltpu.VMEM((NBP,), F32),
            pltpu.VMEM((512,), F32),
            pltpu.VMEM((512,), F32),
            pltpu.VMEM((512,), F32),
            pltpu.VMEM((512,), F32),
            pltpu.VMEM((16,), F32),
        ],
    )
    return kfn(data, perm)


def kernel(features, features_aug):
    cm = jnp.asarray(_CM)
    sm = jnp.asarray(_SM)
    chh = jnp.asarray(_CM[:, :W2])
    shh = jnp.asarray(_SM[:, :W2])
    sum_mag_f, chan_sum_f = _tc_stage(features, cm, sm, chh, shh)
    sum_mag_a, chan_sum_a = _tc_stage(features_aug, cm, sm, chh, shh)
    chan_sum = jnp.stack(
        [chan_sum_f, chan_sum_a], axis=1).reshape(8, N2)
    sum_mag = jnp.stack(
        [sum_mag_f, sum_mag_a], axis=1).reshape(8, N2)
    data = jnp.concatenate([chan_sum, sum_mag], axis=0)
    perm = jnp.asarray(_PERM)
    out = _sc_stage(data, perm)
    return (out[0] + out[16] + out[32] + out[48]) * F32(0.25)


# speedup vs baseline: 1.6551x; 1.6551x over previous
import functools

import numpy as np
import jax
import jax.numpy as jnp
from jax import lax
from jax.experimental import pallas as pl
from jax.experimental.pallas import tpu as pltpu
from jax.experimental.pallas import tpu_sc as plsc

H = W = 224
N = H * W
W2 = W // 2 + 1
H2 = 2 * W2
N2 = H2 * W2
NROW = 25600
KR = 8
NBLK = N // 512
NBP = 112
CB = 8
SUP = 7168
NSUP = N // SUP
F32 = jnp.float32
I32 = jnp.int32


def _dft_mats():
    k = np.arange(H, dtype=np.float64)
    ang = -2.0 * np.pi * np.outer(k, k) / H
    return np.cos(ang).astype(np.float32), np.sin(ang).astype(np.float32)


def _perm_unshifted():
    cy, cx = H // 2, W // 2
    y = np.arange(H, dtype=np.float32) - cy
    x = np.arange(W, dtype=np.float32) - cx
    yy, xx = np.meshgrid(y, x, indexing="ij")
    dist = np.sqrt(xx * xx + yy * yy).astype(np.float32)
    sorted_idx = np.argsort(dist.reshape(-1), kind="stable")
    r, c = sorted_idx // W, sorted_idx % W
    ru, cu = (r + cy) % H, (c + cx) % W
    k2 = np.where(cu <= W // 2, ru, (H - ru) % H)
    l2 = np.where(cu <= W // 2, cu, W - cu)
    sidx = np.where(k2 <= H // 2, k2 * W2 + l2,
                    (W2 + (H - k2)) * W2 + l2)
    return sidx.astype(np.int32)


_CM, _SM = _dft_mats()
_PERM = _perm_unshifted()


def _tc_body(x_ref, chs_ref, ch_ref, sh_ref, summag_ref, chansum_ref):
    j = pl.program_id(1)

    @pl.when(j == 0)
    def _init():
        summag_ref[...] = jnp.zeros_like(summag_ref)
        chansum_ref[...] = jnp.zeros_like(chansum_ref)

    chs = chs_ref[...]
    ch = ch_ref[...]
    sh = sh_ref[...]
    smt = jnp.zeros((W2, W2), F32)
    smb = jnp.zeros((W2, W2), F32)
    cst = jnp.zeros((W2, W2), F32)
    csb = jnp.zeros((W2, W2), F32)
    dot = functools.partial(jnp.dot, preferred_element_type=F32,
                            precision=lax.Precision.HIGHEST)
    xs = x_ref[0].reshape(CB * H, W)
    pq = dot(xs, chs)
    for c in range(CB):
        pqc = pq[c * H:(c + 1) * H]
        cd = dot(ch, pqc)
        sd = dot(sh, pqc)
        u = cd[:, :W2]
        xx = cd[:, W2:]
        y = sd[:, :W2]
        v = sd[:, W2:]
        ret = u - v
        imt = xx + y
        reb = u + v
        imb = xx - y
        mag_t = jnp.sqrt(ret * ret + imt * imt)
        mag_b = jnp.sqrt(reb * reb + imb * imb)
        smt = smt + mag_t
        smb = smb + mag_b
        cst = cst + jnp.log(1.0 + mag_t)
        csb = csb + jnp.log(1.0 + mag_b)
    summag_ref[0, :W2, :] += smt
    summag_ref[0, W2:, :] += smb
    chansum_ref[0, :W2, :] += cst
    chansum_ref[0, W2:, :] += csb


def _tc_stage(x4, chs, ch, sh):
    grid = (4, 96 // CB)
    return pl.pallas_call(
        _tc_body,
        grid=grid,
        in_specs=[
            pl.BlockSpec((1, CB, H, W), lambda s, j: (s, j, 0, 0)),
            pl.BlockSpec((H, H2), lambda s, j: (0, 0)),
            pl.BlockSpec((W2, H), lambda s, j: (0, 0)),
            pl.BlockSpec((W2, H), lambda s, j: (0, 0)),
        ],
        out_specs=[
            pl.BlockSpec((1, H2, W2), lambda s, j: (s, 0, 0)),
            pl.BlockSpec((1, H2, W2), lambda s, j: (s, 0, 0)),
        ],
        out_shape=[
            jax.ShapeDtypeStruct((4, H2, W2), F32),
            jax.ShapeDtypeStruct((4, H2, W2), F32),
        ],
        compiler_params=pltpu.CompilerParams(
            dimension_semantics=("arbitrary", "arbitrary")),
    )(x4, chs, ch, sh)


def _lane_iota():
    return lax.iota(I32, 16)


def _splat(x):
    return jnp.full((16,), x)


def _lane_sum(v):
    return plsc.cumsum(v)[15]


def _sc_body(data_hbm, perm_hbm, out_hbm,
             sorted_sh, part_sh,
             src_v, idx_v, sbuf_v, lp_v,
             p_cf, p_ca, p_mf, p_ma,
             bcf_v, bca_v, bavg_v, cum_v,
             blkA_v, blkB_v, blkC_v, blkD_v, outv_v):
    core = lax.axis_index("c")
    sub = lax.axis_index("s")
    iota = _lane_iota()
    zero16 = jnp.zeros((16,), F32)
    scale_v = jnp.full((16,), F32(0.5 / 96.0))

    @pl.when(sub < 8)
    def _phase_a():
        samp_loc = sub // 4
        a = sub % 4
        s_glob = 2 * (core * 2 + samp_loc) + (a % 2)
        row = jnp.where(a < 2, s_glob, 8 + s_glob)
        lr = samp_loc * 4 + a
        pltpu.sync_copy(data_hbm.at[row], src_v)

        def super_body(sc_i, carry):
            pltpu.sync_copy(perm_hbm.at[pl.ds(sc_i * SUP, SUP)], idx_v)

            def blk_body(jb, carry2):
                def vec_body(i, acc):
                    off = jb * 512 + i * 16
                    iv = idx_v[pl.ds(off, 16)]
                    v = plsc.load_gather(src_v, [iv])
                    sbuf_v[pl.ds(off, 16)] = v
                    return acc + v
                acc = lax.fori_loop(0, 32, vec_body, zero16)
                lp_v[pl.ds((sc_i * 14 + jb) * 16, 16)] = acc
                return carry2
            lax.fori_loop(0, 14, blk_body, 0)
            pltpu.sync_copy(sbuf_v, sorted_sh.at[lr, pl.ds(sc_i * SUP, SUP)])
            return carry
        lax.fori_loop(0, NSUP, super_body, 0)
        for q in range(16 * NBLK, 16 * NBP, 16):
            lp_v[pl.ds(q, 16)] = zero16
        pltpu.sync_copy(lp_v, part_sh.at[lr])

    plsc.subcore_barrier()

    @pl.when(sub < 2)
    def _phase_b():
        samp_loc = sub
        base = samp_loc * 4
        pltpu.sync_copy(part_sh.at[base + 0], p_cf)
        pltpu.sync_copy(part_sh.at[base + 1], p_ca)
        pltpu.sync_copy(part_sh.at[base + 2], p_mf)
        pltpu.sync_copy(part_sh.at[base + 3], p_ma)

        def bs_vec(p_ref, g):
            bidx = (g * 16 + iota) * 16
            acc = zero16
            for l in range(16):
                acc = acc + plsc.load_gather(p_ref, [bidx + l])
            return acc

        for g in range(7):
            bcf_v[pl.ds(g * 16, 16)] = bs_vec(p_cf, g)
            bca_v[pl.ds(g * 16, 16)] = bs_vec(p_ca, g)
            bavg_v[pl.ds(g * 16, 16)] = (
                (bs_vec(p_mf, g) + bs_vec(p_ma, g)) * scale_v)

        carry_v = zero16
        for g in range(7):
            pf = carry_v + plsc.cumsum(bavg_v[pl.ds(g * 16, 16)])
            cum_v[pl.ds(g * 16, 16)] = pf
            carry_v = _splat(pf[15])
        tot_s = cum_v[pl.ds(96, 16)][NBLK - 1 - 96]
        tgt_v = (jnp.full((16,), tot_s) + F32(1e-12)) / F32(KR)

        ends = []
        start_s = jnp.int32(0)
        for kk in range(KR):
            tc_v = F32(kk + 1) * tgt_v
            cnt_s = jnp.int32(0)
            for g in range(7):
                lt = cum_v[pl.ds(g * 16, 16)] < tc_v
                cnt_s = cnt_s + plsc.all_reduce_population_count(lt)[0]
            jb = cnt_s
            jbc = jnp.minimum(jb, NBLK - 1)
            pltpu.sync_copy(sorted_sh.at[base + 2, pl.ds(jbc * 512, 512)],
                            blkA_v)
            pltpu.sync_copy(sorted_sh.at[base + 3, pl.ds(jbc * 512, 512)],
                            blkB_v)
            jb_v = _splat(jb)
            pref_acc = zero16
            for g in range(7):
                jv = _splat(g * 16) + iota
                pref_acc = pref_acc + jnp.where(
                    jv < jb_v, bavg_v[pl.ds(g * 16, 16)], 0.0)
            pref_v = _splat(_lane_sum(pref_acc))

            def rbody(i, c2):
                cv, pv = c2
                w = (blkA_v[pl.ds(i * 16, 16)]
                     + blkB_v[pl.ds(i * 16, 16)]) * scale_v
                pf = cv + plsc.cumsum(w)
                pv = pv + jnp.where(pf < tc_v, 1, 0)
                return (_splat(pf[15]), pv)
            _, pos_v = lax.fori_loop(0, 32, rbody,
                                     (pref_v, jnp.zeros((16,), I32)))
            pos_s = _lane_sum_i32(pos_v)
            end_s = jnp.where(jb >= NBLK, jnp.int32(N),
                              jb * 512 + pos_s)
            end_s = jnp.minimum(jnp.maximum(end_s, start_s + 1),
                                jnp.int32(N))
            if kk == KR - 1:
                end_s = jnp.int32(N)
            ends.append(end_s)
            start_s = end_s

        num_v = zero16
        den_v = zero16
        prev_s = jnp.int32(0)
        for kk in range(KR):
            e_s = ends[kk]
            cnt_i = e_s - prev_s
            jb0 = prev_s // 512
            r0 = prev_s - jb0 * 512
            jb1 = e_s // 512
            r1 = e_s - jb1 * 512
            jb0c = jnp.minimum(jb0, NBLK - 1)
            jb1c = jnp.minimum(jb1, NBLK - 1)
            pltpu.sync_copy(sorted_sh.at[base + 0, pl.ds(jb0c * 512, 512)],
                            blkA_v)
            pltpu.sync_copy(sorted_sh.at[base + 0, pl.ds(jb1c * 512, 512)],
                            blkB_v)
            pltpu.sync_copy(sorted_sh.at[base + 1, pl.ds(jb0c * 512, 512)],
                            blkC_v)
            pltpu.sync_copy(sorted_sh.at[base + 1, pl.ds(jb1c * 512, 512)],
                            blkD_v)
            nsame_v = _splat(jb1) > _splat(jb0)
            r0_v = _splat(r0)
            r1_v = _splat(r1)

            def ebody2(i, accs):
                aLf, aRf, aLa, aRa = accs
                li = _splat(i) * 16 + iota
                lm = (li >= r0_v) & (nsame_v | (li < r1_v))
                rm = (li < r1_v) & nsame_v
                aLf = aLf + jnp.where(lm, blkA_v[pl.ds(i * 16, 16)], 0.0)
                aRf = aRf + jnp.where(rm, blkB_v[pl.ds(i * 16, 16)], 0.0)
                aLa = aLa + jnp.where(lm, blkC_v[pl.ds(i * 16, 16)], 0.0)
                aRa = aRa + jnp.where(rm, blkD_v[pl.ds(i * 16, 16)], 0.0)
                return (aLf, aRf, aLa, aRa)
            aLf, aRf, aLa, aRa = lax.fori_loop(
                0, 32, ebody2, (zero16, zero16, zero16, zero16))

            jb0_v = _splat(jb0)
            jb1_v = _splat(jb1)
            midf = zero16
            mida = zero16
            for g in range(7):
                jv = _splat(g * 16) + iota
                mm = (jv > jb0_v) & (jv < jb1_v)
                midf = midf + jnp.where(mm, bcf_v[pl.ds(g * 16, 16)], 0.0)
                mida = mida + jnp.where(mm, bca_v[pl.ds(g * 16, 16)], 0.0)

            rsf_s = _lane_sum(aLf + aRf + midf)
            rsa_s = _lane_sum(aLa + aRa + mida)
            cnt_v = _splat(cnt_i)
            denom_v = F32(96.0) * jnp.maximum(cnt_v, 1).astype(F32)
            p_v = jnp.full((16,), rsf_s) / denom_v
            pa_v = jnp.full((16,), rsa_s) / denom_v
            valid = cnt_v > 0
            d_v = p_v - pa_v
            num_v = num_v + jnp.where(valid, d_v * d_v, 0.0)
            den_v = den_v + jnp.where(valid, 1.0, 0.0)
            prev_s = e_s

        outv_v[...] = num_v / den_v
        pltpu.sync_copy(
            outv_v, out_hbm.at[pl.ds((core * 2 + samp_loc) * 16, 16)])


def _lane_sum_i32(v):
    return plsc.cumsum(v)[15]


def _sc_stage(data, perm):
    mesh = plsc.VectorSubcoreMesh(core_axis_name="c", subcore_axis_name="s")
    kfn = pl.kernel(
        _sc_body,
        mesh=mesh,
        out_type=jax.ShapeDtypeStruct((64,), F32),
        compiler_params=pltpu.CompilerParams(needs_layout_passes=False),
        scratch_types=[
            pltpu.VMEM_SHARED((8, N), F32),
            pltpu.VMEM_SHARED((8, 16 * NBP), F32),
            pltpu.VMEM((NROW,), F32),
            pltpu.VMEM((SUP,), I32),
            pltpu.VMEM((SUP,), F32),
            pltpu.VMEM((16 * NBP,), F32),
            pltpu.VMEM((16 * NBP,), F32),
            pltpu.VMEM((16 * NBP,), F32),
            pltpu.VMEM((16 * NBP,), F32),
            pltpu.VMEM((16 * NBP,), F32),
            pltpu.VMEM((NBP,), F32),
            pltpu.VMEM((NBP,), F32),
            pltpu.VMEM((NBP,), F32),
            pltpu.VMEM((NBP,), F32),
            pltpu.VMEM((512,), F32),
            pltpu.VMEM((512,), F32),
            pltpu.VMEM((512,), F32),
            pltpu.VMEM((512,), F32),
            pltpu.VMEM((16,), F32),
        ],
    )
    return kfn(data, perm)


def kernel(features, features_aug):
    chs = jnp.asarray(np.concatenate([_CM[:, :W2], _SM[:, :W2]], axis=1))
    ch = jnp.asarray(_CM[:W2, :])
    sh = jnp.asarray(_SM[:W2, :])
    sum_mag_f, chan_sum_f = _tc_stage(features, chs, ch, sh)
    sum_mag_a, chan_sum_a = _tc_stage(features_aug, chs, ch, sh)
    pad = ((0, 0), (0, NROW - N2))
    chan_sum = jnp.pad(jnp.stack(
        [chan_sum_f, chan_sum_a], axis=1).reshape(8, N2), pad)
    sum_mag = jnp.pad(jnp.stack(
        [sum_mag_f, sum_mag_a], axis=1).reshape(8, N2), pad)
    data = jnp.concatenate([chan_sum, sum_mag], axis=0)
    perm = jnp.asarray(_PERM)
    out = _sc_stage(data, perm)
    return (out[0] + out[16] + out[32] + out[48]) * F32(0.25)


# speedup vs baseline: 1.6834x; 1.0171x over previous
import functools

import numpy as np
import jax
import jax.numpy as jnp
from jax import lax
from jax.experimental import pallas as pl
from jax.experimental.pallas import tpu as pltpu
from jax.experimental.pallas import tpu_sc as plsc

H = W = 224
N = H * W
W2 = W // 2 + 1
H2 = 2 * W2
N2 = H2 * W2
NROW = 25600
KR = 8
NBLK = N // 512
NBP = 112
CB = 16
SUP = 7168
NSUP = N // SUP
F32 = jnp.float32
I32 = jnp.int32


def _dft_mats():
    k = np.arange(H, dtype=np.float64)
    ang = -2.0 * np.pi * np.outer(k, k) / H
    return np.cos(ang).astype(np.float32), np.sin(ang).astype(np.float32)


def _perm_unshifted():
    cy, cx = H // 2, W // 2
    y = np.arange(H, dtype=np.float32) - cy
    x = np.arange(W, dtype=np.float32) - cx
    yy, xx = np.meshgrid(y, x, indexing="ij")
    dist = np.sqrt(xx * xx + yy * yy).astype(np.float32)
    sorted_idx = np.argsort(dist.reshape(-1), kind="stable")
    r, c = sorted_idx // W, sorted_idx % W
    ru, cu = (r + cy) % H, (c + cx) % W
    k2 = np.where(cu <= W // 2, ru, (H - ru) % H)
    l2 = np.where(cu <= W // 2, cu, W - cu)
    sidx = np.where(k2 <= H // 2, k2 * W2 + l2,
                    (W2 + (H - k2)) * W2 + l2)
    return sidx.astype(np.int32)


_CM, _SM = _dft_mats()
_PERM = _perm_unshifted()


def _tc_body(x_ref, chs_ref, ch_ref, sh_ref, summag_ref, chansum_ref):
    j = pl.program_id(1)

    @pl.when(j == 0)
    def _init():
        summag_ref[...] = jnp.zeros_like(summag_ref)
        chansum_ref[...] = jnp.zeros_like(chansum_ref)

    chs = chs_ref[...]
    ch = ch_ref[...]
    sh = sh_ref[...]
    smt = jnp.zeros((W2, W2), F32)
    smb = jnp.zeros((W2, W2), F32)
    cst = jnp.zeros((W2, W2), F32)
    csb = jnp.zeros((W2, W2), F32)
    dot = functools.partial(jnp.dot, preferred_element_type=F32,
                            precision=lax.Precision.HIGHEST)
    xs = x_ref[0].reshape(CB * H, W)
    pq = dot(xs, chs)
    for c in range(CB):
        pqc = pq[c * H:(c + 1) * H]
        cd = dot(ch, pqc)
        sd = dot(sh, pqc)
        u = cd[:, :W2]
        xx = cd[:, W2:]
        y = sd[:, :W2]
        v = sd[:, W2:]
        ret = u - v
        imt = xx + y
        reb = u + v
        imb = xx - y
        mag_t = jnp.sqrt(ret * ret + imt * imt)
        mag_b = jnp.sqrt(reb * reb + imb * imb)
        smt = smt + mag_t
        smb = smb + mag_b
        cst = cst + jnp.log(1.0 + mag_t)
        csb = csb + jnp.log(1.0 + mag_b)
    summag_ref[0, :W2, :] += smt
    summag_ref[0, W2:, :] += smb
    chansum_ref[0, :W2, :] += cst
    chansum_ref[0, W2:, :] += csb


def _tc_stage(x4, chs, ch, sh):
    grid = (4, 96 // CB)
    return pl.pallas_call(
        _tc_body,
        grid=grid,
        in_specs=[
            pl.BlockSpec((1, CB, H, W), lambda s, j: (s, j, 0, 0)),
            pl.BlockSpec((H, H2), lambda s, j: (0, 0)),
            pl.BlockSpec((W2, H), lambda s, j: (0, 0)),
            pl.BlockSpec((W2, H), lambda s, j: (0, 0)),
        ],
        out_specs=[
            pl.BlockSpec((1, H2, W2), lambda s, j: (s, 0, 0)),
            pl.BlockSpec((1, H2, W2), lambda s, j: (s, 0, 0)),
        ],
        out_shape=[
            jax.ShapeDtypeStruct((4, H2, W2), F32),
            jax.ShapeDtypeStruct((4, H2, W2), F32),
        ],
        compiler_params=pltpu.CompilerParams(
            dimension_semantics=("arbitrary", "arbitrary")),
    )(x4, chs, ch, sh)


def _lane_iota():
    return lax.iota(I32, 16)


def _splat(x):
    return jnp.full((16,), x)


def _lane_sum(v):
    return plsc.cumsum(v)[15]


def _sc_body(data_hbm, perm_hbm, out_hbm,
             sorted_sh, part_sh,
             src_v, idx_v, sbuf_v, lp_v,
             p_cf, p_ca, p_mf, p_ma,
             bcf_v, bca_v, bavg_v, cum_v,
             blkA_v, blkB_v, blkC_v, blkD_v, outv_v):
    core = lax.axis_index("c")
    sub = lax.axis_index("s")
    iota = _lane_iota()
    zero16 = jnp.zeros((16,), F32)
    scale_v = jnp.full((16,), F32(0.5 / 96.0))

    @pl.when(sub < 8)
    def _phase_a():
        samp_loc = sub // 4
        a = sub % 4
        s_glob = 2 * (core * 2 + samp_loc) + (a % 2)
        row = jnp.where(a < 2, s_glob, 8 + s_glob)
        lr = samp_loc * 4 + a
        pltpu.sync_copy(data_hbm.at[row], src_v)

        def super_body(sc_i, carry):
            pltpu.sync_copy(perm_hbm.at[pl.ds(sc_i * SUP, SUP)], idx_v)

            def blk_body(jb, carry2):
                def vec_body(i, acc):
                    off = jb * 512 + i * 16
                    iv = idx_v[pl.ds(off, 16)]
                    v = plsc.load_gather(src_v, [iv])
                    sbuf_v[pl.ds(off, 16)] = v
                    return acc + v
                acc = lax.fori_loop(0, 32, vec_body, zero16)
                lp_v[pl.ds((sc_i * 14 + jb) * 16, 16)] = acc
                return carry2
            lax.fori_loop(0, 14, blk_body, 0)
            pltpu.sync_copy(sbuf_v, sorted_sh.at[lr, pl.ds(sc_i * SUP, SUP)])
            return carry
        lax.fori_loop(0, NSUP, super_body, 0)
        for q in range(16 * NBLK, 16 * NBP, 16):
            lp_v[pl.ds(q, 16)] = zero16
        pltpu.sync_copy(lp_v, part_sh.at[lr])

    plsc.subcore_barrier()

    @pl.when(sub < 2)
    def _phase_b():
        samp_loc = sub
        base = samp_loc * 4
        pltpu.sync_copy(part_sh.at[base + 0], p_cf)
        pltpu.sync_copy(part_sh.at[base + 1], p_ca)
        pltpu.sync_copy(part_sh.at[base + 2], p_mf)
        pltpu.sync_copy(part_sh.at[base + 3], p_ma)

        def bs_vec(p_ref, g):
            bidx = (g * 16 + iota) * 16
            acc = zero16
            for l in range(16):
                acc = acc + plsc.load_gather(p_ref, [bidx + l])
            return acc

        for g in range(7):
            bcf_v[pl.ds(g * 16, 16)] = bs_vec(p_cf, g)
            bca_v[pl.ds(g * 16, 16)] = bs_vec(p_ca, g)
            bavg_v[pl.ds(g * 16, 16)] = (
                (bs_vec(p_mf, g) + bs_vec(p_ma, g)) * scale_v)

        carry_v = zero16
        for g in range(7):
            pf = carry_v + plsc.cumsum(bavg_v[pl.ds(g * 16, 16)])
            cum_v[pl.ds(g * 16, 16)] = pf
            carry_v = _splat(pf[15])
        tot_s = cum_v[pl.ds(96, 16)][NBLK - 1 - 96]
        tgt_v = (jnp.full((16,), tot_s) + F32(1e-12)) / F32(KR)

        ends = []
        start_s = jnp.int32(0)
        for kk in range(KR):
            tc_v = F32(kk + 1) * tgt_v
            cnt_s = jnp.int32(0)
            for g in range(7):
                lt = cum_v[pl.ds(g * 16, 16)] < tc_v
                cnt_s = cnt_s + plsc.all_reduce_population_count(lt)[0]
            jb = cnt_s
            jbc = jnp.minimum(jb, NBLK - 1)
            pltpu.sync_copy(sorted_sh.at[base + 2, pl.ds(jbc * 512, 512)],
                            blkA_v)
            pltpu.sync_copy(sorted_sh.at[base + 3, pl.ds(jbc * 512, 512)],
                            blkB_v)
            jb_v = _splat(jb)
            pref_acc = zero16
            for g in range(7):
                jv = _splat(g * 16) + iota
                pref_acc = pref_acc + jnp.where(
                    jv < jb_v, bavg_v[pl.ds(g * 16, 16)], 0.0)
            pref_v = _splat(_lane_sum(pref_acc))

            def rbody(i, c2):
                cv, pv = c2
                w = (blkA_v[pl.ds(i * 16, 16)]
                     + blkB_v[pl.ds(i * 16, 16)]) * scale_v
                pf = cv + plsc.cumsum(w)
                pv = pv + jnp.where(pf < tc_v, 1, 0)
                return (_splat(pf[15]), pv)
            _, pos_v = lax.fori_loop(0, 32, rbody,
                                     (pref_v, jnp.zeros((16,), I32)))
            pos_s = _lane_sum_i32(pos_v)
            end_s = jnp.where(jb >= NBLK, jnp.int32(N),
                              jb * 512 + pos_s)
            end_s = jnp.minimum(jnp.maximum(end_s, start_s + 1),
                                jnp.int32(N))
            if kk == KR - 1:
                end_s = jnp.int32(N)
            ends.append(end_s)
            start_s = end_s

        num_v = zero16
        den_v = zero16
        prev_s = jnp.int32(0)
        for kk in range(KR):
            e_s = ends[kk]
            cnt_i = e_s - prev_s
            jb0 = prev_s // 512
            r0 = prev_s - jb0 * 512
            jb1 = e_s // 512
            r1 = e_s - jb1 * 512
            jb0c = jnp.minimum(jb0, NBLK - 1)
            jb1c = jnp.minimum(jb1, NBLK - 1)
            pltpu.sync_copy(sorted_sh.at[base + 0, pl.ds(jb0c * 512, 512)],
                            blkA_v)
            pltpu.sync_copy(sorted_sh.at[base + 0, pl.ds(jb1c * 512, 512)],
                            blkB_v)
            pltpu.sync_copy(sorted_sh.at[base + 1, pl.ds(jb0c * 512, 512)],
                            blkC_v)
            pltpu.sync_copy(sorted_sh.at[base + 1, pl.ds(jb1c * 512, 512)],
                            blkD_v)
            nsame_v = _splat(jb1) > _splat(jb0)
            r0_v = _splat(r0)
            r1_v = _splat(r1)

            def ebody2(i, accs):
                aLf, aRf, aLa, aRa = accs
                li = _splat(i) * 16 + iota
                lm = (li >= r0_v) & (nsame_v | (li < r1_v))
                rm = (li < r1_v) & nsame_v
                aLf = aLf + jnp.where(lm, blkA_v[pl.ds(i * 16, 16)], 0.0)
                aRf = aRf + jnp.where(rm, blkB_v[pl.ds(i * 16, 16)], 0.0)
                aLa = aLa + jnp.where(lm, blkC_v[pl.ds(i * 16, 16)], 0.0)
                aRa = aRa + jnp.where(rm, blkD_v[pl.ds(i * 16, 16)], 0.0)
                return (aLf, aRf, aLa, aRa)
            aLf, aRf, aLa, aRa = lax.fori_loop(
                0, 32, ebody2, (zero16, zero16, zero16, zero16))

            jb0_v = _splat(jb0)
            jb1_v = _splat(jb1)
            midf = zero16
            mida = zero16
            for g in range(7):
                jv = _splat(g * 16) + iota
                mm = (jv > jb0_v) & (jv < jb1_v)
                midf = midf + jnp.where(mm, bcf_v[pl.ds(g * 16, 16)], 0.0)
                mida = mida + jnp.where(mm, bca_v[pl.ds(g * 16, 16)], 0.0)

            rsf_s = _lane_sum(aLf + aRf + midf)
            rsa_s = _lane_sum(aLa + aRa + mida)
            cnt_v = _splat(cnt_i)
            denom_v = F32(96.0) * jnp.maximum(cnt_v, 1).astype(F32)
            p_v = jnp.full((16,), rsf_s) / denom_v
            pa_v = jnp.full((16,), rsa_s) / denom_v
            valid = cnt_v > 0
            d_v = p_v - pa_v
            num_v = num_v + jnp.where(valid, d_v * d_v, 0.0)
            den_v = den_v + jnp.where(valid, 1.0, 0.0)
            prev_s = e_s

        outv_v[...] = num_v / den_v
        pltpu.sync_copy(
            outv_v, out_hbm.at[pl.ds((core * 2 + samp_loc) * 16, 16)])


def _lane_sum_i32(v):
    return plsc.cumsum(v)[15]


def _sc_stage(data, perm):
    mesh = plsc.VectorSubcoreMesh(core_axis_name="c", subcore_axis_name="s")
    kfn = pl.kernel(
        _sc_body,
        mesh=mesh,
        out_type=jax.ShapeDtypeStruct((64,), F32),
        compiler_params=pltpu.CompilerParams(needs_layout_passes=False),
        scratch_types=[
            pltpu.VMEM_SHARED((8, N), F32),
            pltpu.VMEM_SHARED((8, 16 * NBP), F32),
            pltpu.VMEM((NROW,), F32),
            pltpu.VMEM((SUP,), I32),
            pltpu.VMEM((SUP,), F32),
            pltpu.VMEM((16 * NBP,), F32),
            pltpu.VMEM((16 * NBP,), F32),
            pltpu.VMEM((16 * NBP,), F32),
            pltpu.VMEM((16 * NBP,), F32),
            pltpu.VMEM((16 * NBP,), F32),
            pltpu.VMEM((NBP,), F32),
            pltpu.VMEM((NBP,), F32),
            pltpu.VMEM((NBP,), F32),
            pltpu.VMEM((NBP,), F32),
            pltpu.VMEM((512,), F32),
            pltpu.VMEM((512,), F32),
            pltpu.VMEM((512,), F32),
            pltpu.VMEM((512,), F32),
            pltpu.VMEM((16,), F32),
        ],
    )
    return kfn(data, perm)


def kernel(features, features_aug):
    chs = jnp.asarray(np.concatenate([_CM[:, :W2], _SM[:, :W2]], axis=1))
    ch = jnp.asarray(_CM[:W2, :])
    sh = jnp.asarray(_SM[:W2, :])
    sum_mag_f, chan_sum_f = _tc_stage(features, chs, ch, sh)
    sum_mag_a, chan_sum_a = _tc_stage(features_aug, chs, ch, sh)
    pad = ((0, 0), (0, NROW - N2))
    chan_sum = jnp.pad(jnp.stack(
        [chan_sum_f, chan_sum_a], axis=1).reshape(8, N2), pad)
    sum_mag = jnp.pad(jnp.stack(
        [sum_mag_f, sum_mag_a], axis=1).reshape(8, N2), pad)
    data = jnp.concatenate([chan_sum, sum_mag], axis=0)
    perm = jnp.asarray(_PERM)
    out = _sc_stage(data, perm)
    return (out[0] + out[16] + out[32] + out[48]) * F32(0.25)


# speedup vs baseline: 1.7491x; 1.0390x over previous
import functools

import numpy as np
import jax
import jax.numpy as jnp
from jax import lax
from jax.experimental import pallas as pl
from jax.experimental.pallas import tpu as pltpu
from jax.experimental.pallas import tpu_sc as plsc

H = W = 224
N = H * W
W2 = W // 2 + 1
H2 = 2 * W2
N2 = H2 * W2
NROW = 25600
KR = 8
NBLK = N // 512
NBP = 112
CB = 16
SUP = 3584
NSUP = N // SUP
F32 = jnp.float32
I32 = jnp.int32


def _dft_mats():
    k = np.arange(H, dtype=np.float64)
    ang = -2.0 * np.pi * np.outer(k, k) / H
    return np.cos(ang).astype(np.float32), np.sin(ang).astype(np.float32)


def _perm_unshifted():
    cy, cx = H // 2, W // 2
    y = np.arange(H, dtype=np.float32) - cy
    x = np.arange(W, dtype=np.float32) - cx
    yy, xx = np.meshgrid(y, x, indexing="ij")
    dist = np.sqrt(xx * xx + yy * yy).astype(np.float32)
    sorted_idx = np.argsort(dist.reshape(-1), kind="stable")
    r, c = sorted_idx // W, sorted_idx % W
    ru, cu = (r + cy) % H, (c + cx) % W
    k2 = np.where(cu <= W // 2, ru, (H - ru) % H)
    l2 = np.where(cu <= W // 2, cu, W - cu)
    sidx = np.where(k2 <= H // 2, k2 * W2 + l2,
                    (W2 + (H - k2)) * W2 + l2)
    return sidx.astype(np.int32)


_CM, _SM = _dft_mats()
_PERM = _perm_unshifted()


def _tc_body(x_ref, chs_ref, ch_ref, sh_ref, summag_ref, chansum_ref):
    j = pl.program_id(1)

    @pl.when(j == 0)
    def _init():
        summag_ref[...] = jnp.zeros_like(summag_ref)
        chansum_ref[...] = jnp.zeros_like(chansum_ref)

    chs = chs_ref[...]
    ch = ch_ref[...]
    sh = sh_ref[...]
    smt = jnp.zeros((W2, W2), F32)
    smb = jnp.zeros((W2, W2), F32)
    cst = jnp.zeros((W2, W2), F32)
    csb = jnp.zeros((W2, W2), F32)
    dot = functools.partial(jnp.dot, preferred_element_type=F32,
                            precision=lax.Precision.HIGHEST)
    xs = x_ref[0].reshape(CB * H, W)
    pq = dot(xs, chs)
    for c in range(CB):
        pqc = pq[c * H:(c + 1) * H]
        cd = dot(ch, pqc)
        sd = dot(sh, pqc)
        u = cd[:, :W2]
        xx = cd[:, W2:]
        y = sd[:, :W2]
        v = sd[:, W2:]
        ret = u - v
        imt = xx + y
        reb = u + v
        imb = xx - y
        mag_t = jnp.sqrt(ret * ret + imt * imt)
        mag_b = jnp.sqrt(reb * reb + imb * imb)
        smt = smt + mag_t
        smb = smb + mag_b
        cst = cst + jnp.log(1.0 + mag_t)
        csb = csb + jnp.log(1.0 + mag_b)
    summag_ref[0, :W2, :] += smt
    summag_ref[0, W2:, :] += smb
    chansum_ref[0, :W2, :] += cst
    chansum_ref[0, W2:, :] += csb


def _tc_stage(x4, chs, ch, sh):
    grid = (4, 96 // CB)
    return pl.pallas_call(
        _tc_body,
        grid=grid,
        in_specs=[
            pl.BlockSpec((1, CB, H, W), lambda s, j: (s, j, 0, 0)),
            pl.BlockSpec((H, H2), lambda s, j: (0, 0)),
            pl.BlockSpec((W2, H), lambda s, j: (0, 0)),
            pl.BlockSpec((W2, H), lambda s, j: (0, 0)),
        ],
        out_specs=[
            pl.BlockSpec((1, H2, W2), lambda s, j: (s, 0, 0)),
            pl.BlockSpec((1, H2, W2), lambda s, j: (s, 0, 0)),
        ],
        out_shape=[
            jax.ShapeDtypeStruct((4, H2, W2), F32),
            jax.ShapeDtypeStruct((4, H2, W2), F32),
        ],
        compiler_params=pltpu.CompilerParams(
            dimension_semantics=("arbitrary", "arbitrary")),
    )(x4, chs, ch, sh)


def _lane_iota():
    return lax.iota(I32, 16)


def _splat(x):
    return jnp.full((16,), x)


def _lane_sum(v):
    return plsc.cumsum(v)[15]


def _sc_body(data_hbm, perm_hbm, out_hbm,
             sorted_sh, part_sh,
             src_v, idx_v, sbuf_v, lp_v,
             p_cf, p_ca, p_mf, p_ma,
             bcf_v, bca_v, bavg_v, cum_v,
             blkA_v, blkB_v, blkC_v, blkD_v, outv_v):
    core = lax.axis_index("c")
    sub = lax.axis_index("s")
    iota = _lane_iota()
    zero16 = jnp.zeros((16,), F32)
    scale_v = jnp.full((16,), F32(0.5 / 96.0))

    def _phase_a():
        job = sub % 8
        half = sub // 8
        samp_loc = job // 4
        a = job % 4
        s_glob = 2 * (core * 2 + samp_loc) + (a % 2)
        row = jnp.where(a < 2, s_glob, 8 + s_glob)
        lr = samp_loc * 4 + a
        pltpu.sync_copy(data_hbm.at[row], src_v)

        def super_body(sc_i, carry):
            pltpu.sync_copy(perm_hbm.at[pl.ds(sc_i * SUP, SUP)], idx_v)

            def blk_body(jb, carry2):
                def vec_body(i, acc):
                    off = jb * 512 + i * 16
                    iv = idx_v[pl.ds(off, 16)]
                    v = plsc.load_gather(src_v, [iv])
                    sbuf_v[pl.ds(off, 16)] = v
                    return acc + v
                acc = lax.fori_loop(0, 32, vec_body, zero16)
                lp_v[pl.ds(((sc_i - half * 7) * 7 + jb) * 16, 16)] = acc
                return carry2
            lax.fori_loop(0, 7, blk_body, 0)
            pltpu.sync_copy(sbuf_v, sorted_sh.at[lr, pl.ds(sc_i * SUP, SUP)])
            return carry
        lax.fori_loop(half * 7, half * 7 + 7, super_body, 0)
        for q in range(49 * 16, 56 * 16, 16):
            lp_v[pl.ds(q, 16)] = zero16
        pltpu.sync_copy(lp_v, part_sh.at[lr, pl.ds(half * 896, 896)])
    _phase_a()

    plsc.subcore_barrier()

    @pl.when(sub < 2)
    def _phase_b():
        samp_loc = sub
        base = samp_loc * 4
        for pr, lrr in ((p_cf, base + 0), (p_ca, base + 1),
                        (p_mf, base + 2), (p_ma, base + 3)):
            pltpu.sync_copy(part_sh.at[lrr], pr.at[pl.ds(0, 1792)])
            for q in range(1792, 1920, 16):
                pr[pl.ds(q, 16)] = zero16

        def bs_vec(p_ref, g):
            bj = g * 16 + iota
            bidx = bj * 16 + jnp.where(bj >= 49, 112, 0)
            acc = zero16
            for l in range(16):
                acc = acc + plsc.load_gather(p_ref, [bidx + l])
            return acc

        for g in range(7):
            bcf_v[pl.ds(g * 16, 16)] = bs_vec(p_cf, g)
            bca_v[pl.ds(g * 16, 16)] = bs_vec(p_ca, g)
            bavg_v[pl.ds(g * 16, 16)] = (
                (bs_vec(p_mf, g) + bs_vec(p_ma, g)) * scale_v)

        carry_v = zero16
        for g in range(7):
            pf = carry_v + plsc.cumsum(bavg_v[pl.ds(g * 16, 16)])
            cum_v[pl.ds(g * 16, 16)] = pf
            carry_v = _splat(pf[15])
        tot_s = cum_v[pl.ds(96, 16)][NBLK - 1 - 96]
        tgt_v = (jnp.full((16,), tot_s) + F32(1e-12)) / F32(KR)

        ends = []
        start_s = jnp.int32(0)
        for kk in range(KR):
            tc_v = F32(kk + 1) * tgt_v
            cnt_s = jnp.int32(0)
            for g in range(7):
                lt = cum_v[pl.ds(g * 16, 16)] < tc_v
                cnt_s = cnt_s + plsc.all_reduce_population_count(lt)[0]
            jb = cnt_s
            jbc = jnp.minimum(jb, NBLK - 1)
            pltpu.sync_copy(sorted_sh.at[base + 2, pl.ds(jbc * 512, 512)],
                            blkA_v)
            pltpu.sync_copy(sorted_sh.at[base + 3, pl.ds(jbc * 512, 512)],
                            blkB_v)
            jb_v = _splat(jb)
            pref_acc = zero16
            for g in range(7):
                jv = _splat(g * 16) + iota
                pref_acc = pref_acc + jnp.where(
                    jv < jb_v, bavg_v[pl.ds(g * 16, 16)], 0.0)
            pref_v = _splat(_lane_sum(pref_acc))

            def rbody(i, c2):
                cv, pv = c2
                w = (blkA_v[pl.ds(i * 16, 16)]
                     + blkB_v[pl.ds(i * 16, 16)]) * scale_v
                pf = cv + plsc.cumsum(w)
                pv = pv + jnp.where(pf < tc_v, 1, 0)
                return (_splat(pf[15]), pv)
            _, pos_v = lax.fori_loop(0, 32, rbody,
                                     (pref_v, jnp.zeros((16,), I32)))
            pos_s = _lane_sum_i32(pos_v)
            end_s = jnp.where(jb >= NBLK, jnp.int32(N),
                              jb * 512 + pos_s)
            end_s = jnp.minimum(jnp.maximum(end_s, start_s + 1),
                                jnp.int32(N))
            if kk == KR - 1:
                end_s = jnp.int32(N)
            ends.append(end_s)
            start_s = end_s

        num_v = zero16
        den_v = zero16
        prev_s = jnp.int32(0)
        for kk in range(KR):
            e_s = ends[kk]
            cnt_i = e_s - prev_s
            jb0 = prev_s // 512
            r0 = prev_s - jb0 * 512
            jb1 = e_s // 512
            r1 = e_s - jb1 * 512
            jb0c = jnp.minimum(jb0, NBLK - 1)
            jb1c = jnp.minimum(jb1, NBLK - 1)
            pltpu.sync_copy(sorted_sh.at[base + 0, pl.ds(jb0c * 512, 512)],
                            blkA_v)
            pltpu.sync_copy(sorted_sh.at[base + 0, pl.ds(jb1c * 512, 512)],
                            blkB_v)
            pltpu.sync_copy(sorted_sh.at[base + 1, pl.ds(jb0c * 512, 512)],
                            blkC_v)
            pltpu.sync_copy(sorted_sh.at[base + 1, pl.ds(jb1c * 512, 512)],
                            blkD_v)
            nsame_v = _splat(jb1) > _splat(jb0)
            r0_v = _splat(r0)
            r1_v = _splat(r1)

            def ebody2(i, accs):
                aLf, aRf, aLa, aRa = accs
                li = _splat(i) * 16 + iota
                lm = (li >= r0_v) & (nsame_v | (li < r1_v))
                rm = (li < r1_v) & nsame_v
                aLf = aLf + jnp.where(lm, blkA_v[pl.ds(i * 16, 16)], 0.0)
                aRf = aRf + jnp.where(rm, blkB_v[pl.ds(i * 16, 16)], 0.0)
                aLa = aLa + jnp.where(lm, blkC_v[pl.ds(i * 16, 16)], 0.0)
                aRa = aRa + jnp.where(rm, blkD_v[pl.ds(i * 16, 16)], 0.0)
                return (aLf, aRf, aLa, aRa)
            aLf, aRf, aLa, aRa = lax.fori_loop(
                0, 32, ebody2, (zero16, zero16, zero16, zero16))

            jb0_v = _splat(jb0)
            jb1_v = _splat(jb1)
            midf = zero16
            mida = zero16
            for g in range(7):
                jv = _splat(g * 16) + iota
                mm = (jv > jb0_v) & (jv < jb1_v)
                midf = midf + jnp.where(mm, bcf_v[pl.ds(g * 16, 16)], 0.0)
                mida = mida + jnp.where(mm, bca_v[pl.ds(g * 16, 16)], 0.0)

            rsf_s = _lane_sum(aLf + aRf + midf)
            rsa_s = _lane_sum(aLa + aRa + mida)
            cnt_v = _splat(cnt_i)
            denom_v = F32(96.0) * jnp.maximum(cnt_v, 1).astype(F32)
            p_v = jnp.full((16,), rsf_s) / denom_v
            pa_v = jnp.full((16,), rsa_s) / denom_v
            valid = cnt_v > 0
            d_v = p_v - pa_v
            num_v = num_v + jnp.where(valid, d_v * d_v, 0.0)
            den_v = den_v + jnp.where(valid, 1.0, 0.0)
            prev_s = e_s

        outv_v[...] = num_v / den_v
        pltpu.sync_copy(
            outv_v, out_hbm.at[pl.ds((core * 2 + samp_loc) * 16, 16)])


def _lane_sum_i32(v):
    return plsc.cumsum(v)[15]


def _sc_stage(data, perm):
    mesh = plsc.VectorSubcoreMesh(core_axis_name="c", subcore_axis_name="s")
    kfn = pl.kernel(
        _sc_body,
        mesh=mesh,
        out_type=jax.ShapeDtypeStruct((64,), F32),
        compiler_params=pltpu.CompilerParams(needs_layout_passes=False),
        scratch_types=[
            pltpu.VMEM_SHARED((8, N), F32),
            pltpu.VMEM_SHARED((8, 1792), F32),
            pltpu.VMEM((NROW,), F32),
            pltpu.VMEM((SUP,), I32),
            pltpu.VMEM((SUP,), F32),
            pltpu.VMEM((896,), F32),
            pltpu.VMEM((1920,), F32),
            pltpu.VMEM((1920,), F32),
            pltpu.VMEM((1920,), F32),
            pltpu.VMEM((1920,), F32),
            pltpu.VMEM((NBP,), F32),
            pltpu.VMEM((NBP,), F32),
            pltpu.VMEM((NBP,), F32),
            pltpu.VMEM((NBP,), F32),
            pltpu.VMEM((512,), F32),
            pltpu.VMEM((512,), F32),
            pltpu.VMEM((512,), F32),
            pltpu.VMEM((512,), F32),
            pltpu.VMEM((16,), F32),
        ],
    )
    return kfn(data, perm)


def kernel(features, features_aug):
    chs = jnp.asarray(np.concatenate([_CM[:, :W2], _SM[:, :W2]], axis=1))
    ch = jnp.asarray(_CM[:W2, :])
    sh = jnp.asarray(_SM[:W2, :])
    sum_mag_f, chan_sum_f = _tc_stage(features, chs, ch, sh)
    sum_mag_a, chan_sum_a = _tc_stage(features_aug, chs, ch, sh)
    pad = ((0, 0), (0, NROW - N2))
    chan_sum = jnp.pad(jnp.stack(
        [chan_sum_f, chan_sum_a], axis=1).reshape(8, N2), pad)
    sum_mag = jnp.pad(jnp.stack(
        [sum_mag_f, sum_mag_a], axis=1).reshape(8, N2), pad)
    data = jnp.concatenate([chan_sum, sum_mag], axis=0)
    perm = jnp.asarray(_PERM)
    out = _sc_stage(data, perm)
    return (out[0] + out[16] + out[32] + out[48]) * F32(0.25)
